# Initial kernel scaffold; baseline (speedup 1.0000x reference)
import jax
import jax.numpy as jnp
from jax.experimental import pallas as pl


def kernel(prototypes, x0, W_score, b_score):
    raise NotImplementedError("write your pallas kernel here")


# speedup vs baseline: 19.3563x; 19.3563x over previous
import functools

import jax
import jax.numpy as jnp
import numpy as np
from jax.experimental import pallas as pl


def _merge_body(n_classes, proto_ref, x0_ref, w_ref, b_ref, out_ref, idx_ref):
    p = proto_ref[0]
    P, C = p.shape
    x = jnp.concatenate([p, x0_ref[0]], axis=0)
    N = x.shape[0]
    n_cp = P // n_classes

    p2 = jnp.sum(p * p, axis=1, keepdims=True)
    x2 = jnp.sum(x * x, axis=1, keepdims=True)
    g = jnp.dot(p, x.T, preferred_element_type=jnp.float32)
    d2 = p2 + x2.T - 2.0 * g
    dist = jnp.sqrt(jnp.maximum(d2, 0.0)) * (1.0 / np.sqrt(C))

    m = jnp.min(dist, axis=0, keepdims=True)
    pio = jax.lax.broadcasted_iota(jnp.int32, (P, N), 0)
    idx = jnp.min(jnp.where(dist == m, pio, P), axis=0, keepdims=True)

    e = jnp.exp(-dist)
    c0 = jnp.sum(e[:n_cp], axis=0, keepdims=True) * (1.0 / n_cp)
    c1 = jnp.sum(e[n_cp:], axis=0, keepdims=True) * (1.0 / n_cp)
    tot = c0 + c1 + 1e-6
    cw = jnp.where(idx < n_cp, c0, c1) / tot

    z = jnp.sum(x * w_ref[...], axis=1, keepdims=True).T + b_ref[0, 0]
    sw = 1.0 / (1.0 + jnp.exp(-z))

    mask = (pio == idx).astype(jnp.float32)
    all_c = jnp.sum(mask * cw, axis=1, keepdims=True) + 1e-6
    all_s = jnp.sum(mask * sw, axis=1, keepdims=True) + 1e-6
    g_c = jnp.sum(mask * all_c, axis=0, keepdims=True)
    g_s = jnp.sum(mask * all_s, axis=0, keepdims=True)
    w = 0.5 * cw / g_c + 0.5 * sw / g_s

    out_ref[0] = jnp.dot(mask * w, x, preferred_element_type=jnp.float32)
    idx_ref[0] = idx


def kernel(prototypes, x0, W_score, b_score):
    B, P, C = prototypes.shape
    N0 = x0.shape[1]
    N = P + N0
    n_classes = 2

    w_row = W_score.reshape(1, C)
    b_2d = b_score.reshape(1, 1)

    out, idx3 = pl.pallas_call(
        functools.partial(_merge_body, n_classes),
        grid=(B,),
        in_specs=[
            pl.BlockSpec((1, P, C), lambda b: (b, 0, 0)),
            pl.BlockSpec((1, N0, C), lambda b: (b, 0, 0)),
            pl.BlockSpec((1, C), lambda b: (0, 0)),
            pl.BlockSpec((1, 1), lambda b: (0, 0)),
        ],
        out_specs=[
            pl.BlockSpec((1, P, C), lambda b: (b, 0, 0)),
            pl.BlockSpec((1, 1, N), lambda b: (b, 0, 0)),
        ],
        out_shape=[
            jax.ShapeDtypeStruct((B, P, C), jnp.float32),
            jax.ShapeDtypeStruct((B, 1, N), jnp.int32),
        ],
    )(prototypes, x0, w_row, b_2d)
    return (out, idx3.reshape(B, N))
